# baseline (speedup 1.0000x reference)
import functools

import jax
import jax.numpy as jnp
from jax import lax
from jax.experimental import pallas as pl
from jax.experimental.pallas import tpu as pltpu
from jax.experimental.pallas import tpu_sc as plsc

D = 1024
M = 2048
E = 8
K = 2
T = 2048
TILE = 256
NT = 24
ROWS = NT * TILE

NC = 2
NS = 16
NW = NC * NS
TPW = T // NW
SUB = 16
NSUB = TPW // SUB

_NEG_INF = -1e30


def _dispatch_body(x_ref, rw_ref, p1_ref, p2_ref, g1_ref, g2_ref, tmap_ref):
    x = x_ref[...]
    rw = rw_ref[...]
    logits = lax.dot_general(x, rw, (((1,), (1,)), ((), ())),
                             preferred_element_type=jnp.float32)
    m = jnp.max(logits, axis=1, keepdims=True)
    ex = jnp.exp(logits - m)
    probs = ex / jnp.sum(ex, axis=1, keepdims=True)

    iota_e = lax.broadcasted_iota(jnp.int32, (T, E), 1)
    m1 = jnp.max(probs, axis=1, keepdims=True)
    eq1 = probs >= m1
    idx1 = jnp.min(jnp.where(eq1, iota_e, E), axis=1, keepdims=True)
    masked = jnp.where(iota_e == idx1, _NEG_INF, probs)
    m2 = jnp.max(masked, axis=1, keepdims=True)
    eq2 = masked >= m2
    idx2 = jnp.min(jnp.where(eq2, iota_e, E), axis=1, keepdims=True)

    onehot = ((iota_e == idx1) | (iota_e == idx2)).astype(jnp.float32)

    r_i = lax.broadcasted_iota(jnp.int32, (T, T), 0)
    c_i = lax.broadcasted_iota(jnp.int32, (T, T), 1)
    tril = (c_i < r_i).astype(jnp.float32)
    rank = lax.dot_general(tril, onehot, (((1,), (0,)), ((), ())),
                           preferred_element_type=jnp.float32)
    counts = jnp.sum(onehot, axis=0, keepdims=True)

    tiles = jnp.ceil(counts / TILE)
    e_i = lax.broadcasted_iota(jnp.int32, (E, E), 0)
    f_i = lax.broadcasted_iota(jnp.int32, (E, E), 1)
    tril_e = (e_i < f_i).astype(jnp.float32)
    tile_base = lax.dot_general(tiles, tril_e, (((1,), (0,)), ((), ())),
                                preferred_element_type=jnp.float32) * TILE

    slot = tile_base + rank
    slot_i = slot.astype(jnp.int32)
    p1_ref[...] = jnp.sum(jnp.where(iota_e == idx1, slot_i, 0), axis=1,
                          keepdims=True)
    p2_ref[...] = jnp.sum(jnp.where(iota_e == idx2, slot_i, 0), axis=1,
                          keepdims=True)

    g2 = jnp.sum(jnp.where(iota_e == idx2, probs, 0.0), axis=1, keepdims=True)
    g1_ref[...] = jnp.broadcast_to(m1, (T, 16))
    g2_ref[...] = jnp.broadcast_to(g2, (T, 16))

    cum_incl = lax.dot_general(
        tiles, (e_i <= f_i).astype(jnp.float32), (((1,), (0,)), ((), ())),
        preferred_element_type=jnp.float32)
    t_i = lax.broadcasted_iota(jnp.int32, (NT, E), 0)
    cum_incl_i = jnp.broadcast_to(cum_incl.astype(jnp.int32), (NT, E))
    emap = jnp.sum((cum_incl_i <= t_i).astype(jnp.int32),
                   axis=1, keepdims=True)
    tmap_ref[...] = jnp.minimum(emap, E - 1)


def _dispatch(x, router_w):
    return pl.pallas_call(
        _dispatch_body,
        out_shape=(
            jax.ShapeDtypeStruct((T, 1), jnp.int32),
            jax.ShapeDtypeStruct((T, 1), jnp.int32),
            jax.ShapeDtypeStruct((T, 16), jnp.float32),
            jax.ShapeDtypeStruct((T, 16), jnp.float32),
            jax.ShapeDtypeStruct((NT, 1), jnp.int32),
        ),
    )(x, router_w)


def _sc_scatter_body(x_hbm, p1_hbm, p2_hbm, xs_hbm,
                     rows_a, rows_b, i1a, i2a, i1b, i2b, sem_a, sem_b):
    wid = lax.axis_index("s") * NC + lax.axis_index("c")
    bufs = ((rows_a, i1a, i2a, sem_a), (rows_b, i1b, i2b, sem_b))
    pend = [None, None]
    for s in range(NSUB):
        t0 = wid * TPW + s * SUB
        rows, i1, i2, sem = bufs[s % 2]
        if pend[s % 2] is not None:
            pend[s % 2][0].wait()
            pend[s % 2][1].wait()
        pltpu.sync_copy(x_hbm.at[pl.ds(t0, SUB)], rows)
        pltpu.sync_copy(p1_hbm.at[pl.ds(t0, SUB)], i1)
        pltpu.sync_copy(p2_hbm.at[pl.ds(t0, SUB)], i2)
        c1 = pltpu.async_copy(rows, xs_hbm.at[i1], sem)
        c2 = pltpu.async_copy(rows, xs_hbm.at[i2], sem)
        pend[s % 2] = (c1, c2)
    for p in pend:
        if p is not None:
            p[0].wait()
            p[1].wait()


def _sc_scatter(x, pos1, pos2):
    return pl.kernel(
        _sc_scatter_body,
        out_type=jax.ShapeDtypeStruct((ROWS, D), jnp.float32),
        mesh=plsc.VectorSubcoreMesh(core_axis_name="c", subcore_axis_name="s"),
        scratch_types=[
            pltpu.VMEM((SUB, D), jnp.float32),
            pltpu.VMEM((SUB, D), jnp.float32),
            pltpu.VMEM((SUB,), jnp.int32),
            pltpu.VMEM((SUB,), jnp.int32),
            pltpu.VMEM((SUB,), jnp.int32),
            pltpu.VMEM((SUB,), jnp.int32),
            pltpu.SemaphoreType.DMA,
            pltpu.SemaphoreType.DMA,
        ],
    )(x, pos1, pos2)


def _sc_combine_body(y_hbm, p1_hbm, p2_hbm, g1_hbm, g2_hbm, sh_hbm, out_hbm,
                     y1a, y2a, sha, g1a, g2a, i1a, i2a,
                     y1b, y2b, shb, g1b, g2b, i1b, i2b, sem_a, sem_b):
    wid = lax.axis_index("s") * NC + lax.axis_index("c")
    bufs = ((y1a, y2a, sha, g1a, g2a, i1a, i2a, sem_a),
            (y1b, y2b, shb, g1b, g2b, i1b, i2b, sem_b))
    pend = [None, None]

    def issue(s):
        t0 = wid * TPW + s * SUB
        y1, y2, sh, g1, g2, i1, i2, sem = bufs[s % 2]
        pltpu.sync_copy(p1_hbm.at[pl.ds(t0, SUB)], i1)
        pltpu.sync_copy(p2_hbm.at[pl.ds(t0, SUB)], i2)
        c1 = pltpu.async_copy(y_hbm.at[i1], y1, sem)
        c2 = pltpu.async_copy(y_hbm.at[i2], y2, sem)
        c3 = pltpu.async_copy(sh_hbm.at[pl.ds(t0, SUB)], sh, sem)
        c4 = pltpu.async_copy(g1_hbm.at[pl.ds(t0, SUB)], g1, sem)
        c5 = pltpu.async_copy(g2_hbm.at[pl.ds(t0, SUB)], g2, sem)
        pend[s % 2] = (c1, c2, c3, c4, c5)

    issue(0)
    for s in range(NSUB):
        if s + 1 < NSUB:
            issue(s + 1)
        y1, y2, sh, g1, g2, i1, i2, sem = bufs[s % 2]
        for c in pend[s % 2]:
            c.wait()
        for i in range(SUB):
            g1v = g1[i]
            g2v = g2[i]

            def sl_body(j, _):
                sl = pl.ds(j * 16, 16)
                sh[i, sl] = sh[i, sl] + g1v * y1[i, sl] + g2v * y2[i, sl]
                return 0

            lax.fori_loop(0, D // 16, sl_body, 0, unroll=4)
        t0 = wid * TPW + s * SUB
        pltpu.sync_copy(sh, out_hbm.at[pl.ds(t0, SUB)])


def _sc_combine(y_sorted, pos1, pos2, g1, g2, shared):
    buf = lambda: pltpu.VMEM((SUB, D), jnp.float32)
    return pl.kernel(
        _sc_combine_body,
        out_type=jax.ShapeDtypeStruct((T, D), jnp.float32),
        mesh=plsc.VectorSubcoreMesh(core_axis_name="c", subcore_axis_name="s"),
        scratch_types=[
            buf(), buf(), buf(),
            pltpu.VMEM((SUB, 16), jnp.float32),
            pltpu.VMEM((SUB, 16), jnp.float32),
            pltpu.VMEM((SUB,), jnp.int32),
            pltpu.VMEM((SUB,), jnp.int32),
            buf(), buf(), buf(),
            pltpu.VMEM((SUB, 16), jnp.float32),
            pltpu.VMEM((SUB, 16), jnp.float32),
            pltpu.VMEM((SUB,), jnp.int32),
            pltpu.VMEM((SUB,), jnp.int32),
            pltpu.SemaphoreType.DMA,
            pltpu.SemaphoreType.DMA,
        ],
    )(y_sorted, pos1, pos2, g1, g2, shared)


def _ffn_body(tmap_ref, x_ref, w1_ref, w2_ref, y_ref):
    x = x_ref[...]
    w1 = w1_ref[0]
    inter = lax.dot_general(x, w1, (((1,), (1,)), ((), ())),
                            preferred_element_type=jnp.float32,
                            precision=lax.Precision.DEFAULT)
    h = jnp.square(jnp.maximum(inter, 0.0))
    w2 = w2_ref[0]
    y_ref[...] = lax.dot_general(h, w2, (((1,), (1,)), ((), ())),
                                 preferred_element_type=jnp.float32,
                                 precision=lax.Precision.DEFAULT)


def _expert_ffn(tmap, x_sorted, w1_stack, w2_stack):
    grid_spec = pltpu.PrefetchScalarGridSpec(
        num_scalar_prefetch=1,
        grid=(NT,),
        in_specs=[
            pl.BlockSpec((TILE, D), lambda i, m: (i, 0)),
            pl.BlockSpec((1, M, D), lambda i, m: (m[i, 0], 0, 0)),
            pl.BlockSpec((1, D, M), lambda i, m: (m[i, 0], 0, 0)),
        ],
        out_specs=pl.BlockSpec((TILE, D), lambda i, m: (i, 0)),
    )
    return pl.pallas_call(
        _ffn_body,
        grid_spec=grid_spec,
        out_shape=jax.ShapeDtypeStruct((ROWS, D), jnp.float32),
        compiler_params=pltpu.CompilerParams(
            dimension_semantics=("arbitrary",)),
    )(tmap, x_sorted, w1_stack, w2_stack)


def _shared_body(x_ref, w1_ref, w2_ref, y_ref):
    x = x_ref[...]
    inter = lax.dot_general(x, w1_ref[...], (((1,), (1,)), ((), ())),
                            preferred_element_type=jnp.float32,
                            precision=lax.Precision.DEFAULT)
    h = jnp.square(jnp.maximum(inter, 0.0))
    y_ref[...] = lax.dot_general(h, w2_ref[...], (((1,), (1,)), ((), ())),
                                 preferred_element_type=jnp.float32,
                                 precision=lax.Precision.DEFAULT)


def _shared_ffn(x, shared_w1, shared_w2):
    return pl.pallas_call(
        _shared_body,
        grid=(T // TILE,),
        in_specs=[
            pl.BlockSpec((TILE, D), lambda i: (i, 0)),
            pl.BlockSpec((M, D), lambda i: (0, 0)),
            pl.BlockSpec((D, M), lambda i: (0, 0)),
        ],
        out_specs=pl.BlockSpec((TILE, D), lambda i: (i, 0)),
        out_shape=jax.ShapeDtypeStruct((T, D), jnp.float32),
        compiler_params=pltpu.CompilerParams(
            dimension_semantics=("arbitrary",)),
    )(x, shared_w1, shared_w2)


def kernel(hidden_tensor, router_w, w1_stack, w2_stack, shared_w1, shared_w2):
    b, t, c = hidden_tensor.shape
    x = hidden_tensor.reshape(-1, c)

    pos1, pos2, g1, g2, tmap = _dispatch(x, router_w)
    x_sorted = _sc_scatter(x, pos1.reshape(-1), pos2.reshape(-1))
    shared = _shared_ffn(x, shared_w1, shared_w2)
    y_sorted = _expert_ffn(tmap, x_sorted, w1_stack, w2_stack)
    out = _sc_combine(y_sorted, pos1.reshape(-1), pos2.reshape(-1), g1, g2,
                      shared)
    return out.reshape(b, t, c)


# speedup vs baseline: 1.0089x; 1.0089x over previous
import functools

import jax
import jax.numpy as jnp
from jax import lax
from jax.experimental import pallas as pl
from jax.experimental.pallas import tpu as pltpu
from jax.experimental.pallas import tpu_sc as plsc

D = 1024
M = 2048
E = 8
K = 2
T = 2048
TILE = 256
NT = 24
ROWS = NT * TILE

NC = 2
NS = 16
NW = NC * NS
TPW = T // NW
SUB = 16
NSUB = TPW // SUB

_NEG_INF = -1e30


def _dispatch_body(x_ref, rw_ref, p1_ref, p2_ref, g1_ref, g2_ref, tmap_ref):
    x = x_ref[...]
    rw = rw_ref[...]
    logits = lax.dot_general(x, rw, (((1,), (1,)), ((), ())),
                             preferred_element_type=jnp.float32)
    m = jnp.max(logits, axis=1, keepdims=True)
    ex = jnp.exp(logits - m)
    probs = ex / jnp.sum(ex, axis=1, keepdims=True)

    iota_e = lax.broadcasted_iota(jnp.int32, (T, E), 1)
    m1 = jnp.max(probs, axis=1, keepdims=True)
    eq1 = probs >= m1
    idx1 = jnp.min(jnp.where(eq1, iota_e, E), axis=1, keepdims=True)
    masked = jnp.where(iota_e == idx1, _NEG_INF, probs)
    m2 = jnp.max(masked, axis=1, keepdims=True)
    eq2 = masked >= m2
    idx2 = jnp.min(jnp.where(eq2, iota_e, E), axis=1, keepdims=True)

    onehot = ((iota_e == idx1) | (iota_e == idx2)).astype(jnp.float32)

    GS = 128
    NG = T // GS
    r_g = lax.broadcasted_iota(jnp.int32, (GS, GS), 0)
    c_g = lax.broadcasted_iota(jnp.int32, (GS, GS), 1)
    tril_g = jnp.broadcast_to((c_g < r_g).astype(jnp.float32), (NG, GS, GS))
    oh_r = onehot.reshape(NG, GS, E)
    rank_f = lax.dot_general(tril_g, oh_r, (((2,), (1,)), ((0,), (0,))),
                             preferred_element_type=jnp.float32)
    g_r = lax.broadcasted_iota(jnp.int32, (NG, T), 0)
    g_c = lax.broadcasted_iota(jnp.int32, (NG, T), 1)
    A = (g_c // GS == g_r).astype(jnp.float32)
    grp_counts = lax.dot_general(A, onehot, (((1,), (0,)), ((), ())),
                                 preferred_element_type=jnp.float32)
    n_r = lax.broadcasted_iota(jnp.int32, (NG, NG), 0)
    n_c = lax.broadcasted_iota(jnp.int32, (NG, NG), 1)
    tril_n = (n_c < n_r).astype(jnp.float32)
    grp_base = lax.dot_general(tril_n, grp_counts, (((1,), (0,)), ((), ())),
                               preferred_element_type=jnp.float32)
    base_t = lax.dot_general(A, grp_base, (((0,), (0,)), ((), ())),
                             preferred_element_type=jnp.float32)
    rank = base_t + rank_f.reshape(T, E)
    counts = jnp.sum(grp_counts, axis=0, keepdims=True)

    tiles = jnp.ceil(counts / TILE)
    e_i = lax.broadcasted_iota(jnp.int32, (E, E), 0)
    f_i = lax.broadcasted_iota(jnp.int32, (E, E), 1)
    tril_e = (e_i < f_i).astype(jnp.float32)
    tile_base = lax.dot_general(tiles, tril_e, (((1,), (0,)), ((), ())),
                                preferred_element_type=jnp.float32) * TILE

    slot = tile_base + rank
    slot_i = slot.astype(jnp.int32)
    p1_ref[...] = jnp.sum(jnp.where(iota_e == idx1, slot_i, 0), axis=1,
                          keepdims=True)
    p2_ref[...] = jnp.sum(jnp.where(iota_e == idx2, slot_i, 0), axis=1,
                          keepdims=True)

    g2 = jnp.sum(jnp.where(iota_e == idx2, probs, 0.0), axis=1, keepdims=True)
    g1_ref[...] = jnp.broadcast_to(m1, (T, 16))
    g2_ref[...] = jnp.broadcast_to(g2, (T, 16))

    cum_incl = lax.dot_general(
        tiles, (e_i <= f_i).astype(jnp.float32), (((1,), (0,)), ((), ())),
        preferred_element_type=jnp.float32)
    t_i = lax.broadcasted_iota(jnp.int32, (NT, E), 0)
    cum_incl_i = jnp.broadcast_to(cum_incl.astype(jnp.int32), (NT, E))
    emap = jnp.sum((cum_incl_i <= t_i).astype(jnp.int32),
                   axis=1, keepdims=True)
    tmap_ref[...] = jnp.minimum(emap, E - 1)


def _dispatch(x, router_w):
    return pl.pallas_call(
        _dispatch_body,
        out_shape=(
            jax.ShapeDtypeStruct((T, 1), jnp.int32),
            jax.ShapeDtypeStruct((T, 1), jnp.int32),
            jax.ShapeDtypeStruct((T, 16), jnp.float32),
            jax.ShapeDtypeStruct((T, 16), jnp.float32),
            jax.ShapeDtypeStruct((NT, 1), jnp.int32),
        ),
    )(x, router_w)


def _sc_scatter_body(x_hbm, p1_hbm, p2_hbm, xs_hbm,
                     rows_a, rows_b, i1a, i2a, i1b, i2b, sem_a, sem_b):
    wid = lax.axis_index("s") * NC + lax.axis_index("c")
    bufs = ((rows_a, i1a, i2a, sem_a), (rows_b, i1b, i2b, sem_b))
    pend = [None, None]
    for s in range(NSUB):
        t0 = wid * TPW + s * SUB
        rows, i1, i2, sem = bufs[s % 2]
        if pend[s % 2] is not None:
            pend[s % 2][0].wait()
            pend[s % 2][1].wait()
        pltpu.sync_copy(x_hbm.at[pl.ds(t0, SUB)], rows)
        pltpu.sync_copy(p1_hbm.at[pl.ds(t0, SUB)], i1)
        pltpu.sync_copy(p2_hbm.at[pl.ds(t0, SUB)], i2)
        c1 = pltpu.async_copy(rows, xs_hbm.at[i1], sem)
        c2 = pltpu.async_copy(rows, xs_hbm.at[i2], sem)
        pend[s % 2] = (c1, c2)
    for p in pend:
        if p is not None:
            p[0].wait()
            p[1].wait()


def _sc_scatter(x, pos1, pos2):
    return pl.kernel(
        _sc_scatter_body,
        out_type=jax.ShapeDtypeStruct((ROWS, D), jnp.float32),
        mesh=plsc.VectorSubcoreMesh(core_axis_name="c", subcore_axis_name="s"),
        scratch_types=[
            pltpu.VMEM((SUB, D), jnp.float32),
            pltpu.VMEM((SUB, D), jnp.float32),
            pltpu.VMEM((SUB,), jnp.int32),
            pltpu.VMEM((SUB,), jnp.int32),
            pltpu.VMEM((SUB,), jnp.int32),
            pltpu.VMEM((SUB,), jnp.int32),
            pltpu.SemaphoreType.DMA,
            pltpu.SemaphoreType.DMA,
        ],
    )(x, pos1, pos2)


def _sc_combine_body(y_hbm, p1_hbm, p2_hbm, g1_hbm, g2_hbm, sh_hbm, out_hbm,
                     y1a, y2a, sha, g1a, g2a, i1a, i2a,
                     y1b, y2b, shb, g1b, g2b, i1b, i2b, sem_a, sem_b):
    wid = lax.axis_index("s") * NC + lax.axis_index("c")
    bufs = ((y1a, y2a, sha, g1a, g2a, i1a, i2a, sem_a),
            (y1b, y2b, shb, g1b, g2b, i1b, i2b, sem_b))
    pend = [None, None]

    def issue(s):
        t0 = wid * TPW + s * SUB
        y1, y2, sh, g1, g2, i1, i2, sem = bufs[s % 2]
        pltpu.sync_copy(p1_hbm.at[pl.ds(t0, SUB)], i1)
        pltpu.sync_copy(p2_hbm.at[pl.ds(t0, SUB)], i2)
        c1 = pltpu.async_copy(y_hbm.at[i1], y1, sem)
        c2 = pltpu.async_copy(y_hbm.at[i2], y2, sem)
        c3 = pltpu.async_copy(sh_hbm.at[pl.ds(t0, SUB)], sh, sem)
        c4 = pltpu.async_copy(g1_hbm.at[pl.ds(t0, SUB)], g1, sem)
        c5 = pltpu.async_copy(g2_hbm.at[pl.ds(t0, SUB)], g2, sem)
        pend[s % 2] = (c1, c2, c3, c4, c5)

    issue(0)
    for s in range(NSUB):
        if s + 1 < NSUB:
            issue(s + 1)
        y1, y2, sh, g1, g2, i1, i2, sem = bufs[s % 2]
        for c in pend[s % 2]:
            c.wait()
        for i in range(SUB):
            g1v = g1[i]
            g2v = g2[i]

            def sl_body(j, _):
                sl = pl.ds(j * 16, 16)
                sh[i, sl] = sh[i, sl] + g1v * y1[i, sl] + g2v * y2[i, sl]
                return 0

            lax.fori_loop(0, D // 16, sl_body, 0, unroll=4)
        t0 = wid * TPW + s * SUB
        pltpu.sync_copy(sh, out_hbm.at[pl.ds(t0, SUB)])


def _sc_combine(y_sorted, pos1, pos2, g1, g2, shared):
    buf = lambda: pltpu.VMEM((SUB, D), jnp.float32)
    return pl.kernel(
        _sc_combine_body,
        out_type=jax.ShapeDtypeStruct((T, D), jnp.float32),
        mesh=plsc.VectorSubcoreMesh(core_axis_name="c", subcore_axis_name="s"),
        scratch_types=[
            buf(), buf(), buf(),
            pltpu.VMEM((SUB, 16), jnp.float32),
            pltpu.VMEM((SUB, 16), jnp.float32),
            pltpu.VMEM((SUB,), jnp.int32),
            pltpu.VMEM((SUB,), jnp.int32),
            buf(), buf(), buf(),
            pltpu.VMEM((SUB, 16), jnp.float32),
            pltpu.VMEM((SUB, 16), jnp.float32),
            pltpu.VMEM((SUB,), jnp.int32),
            pltpu.VMEM((SUB,), jnp.int32),
            pltpu.SemaphoreType.DMA,
            pltpu.SemaphoreType.DMA,
        ],
    )(y_sorted, pos1, pos2, g1, g2, shared)


def _ffn_body(tmap_ref, x_ref, w1_ref, w2_ref, y_ref):
    x = x_ref[...]
    w1 = w1_ref[0]
    inter = lax.dot_general(x, w1, (((1,), (1,)), ((), ())),
                            preferred_element_type=jnp.float32,
                            precision=lax.Precision.DEFAULT)
    h = jnp.square(jnp.maximum(inter, 0.0))
    w2 = w2_ref[0]
    y_ref[...] = lax.dot_general(h, w2, (((1,), (1,)), ((), ())),
                                 preferred_element_type=jnp.float32,
                                 precision=lax.Precision.DEFAULT)


def _expert_ffn(tmap, x_sorted, w1_stack, w2_stack):
    grid_spec = pltpu.PrefetchScalarGridSpec(
        num_scalar_prefetch=1,
        grid=(NT,),
        in_specs=[
            pl.BlockSpec((TILE, D), lambda i, m: (i, 0)),
            pl.BlockSpec((1, M, D), lambda i, m: (m[i, 0], 0, 0)),
            pl.BlockSpec((1, D, M), lambda i, m: (m[i, 0], 0, 0)),
        ],
        out_specs=pl.BlockSpec((TILE, D), lambda i, m: (i, 0)),
    )
    return pl.pallas_call(
        _ffn_body,
        grid_spec=grid_spec,
        out_shape=jax.ShapeDtypeStruct((ROWS, D), jnp.float32),
        compiler_params=pltpu.CompilerParams(
            dimension_semantics=("arbitrary",)),
    )(tmap, x_sorted, w1_stack, w2_stack)


def _shared_body(x_ref, w1_ref, w2_ref, y_ref):
    x = x_ref[...]
    inter = lax.dot_general(x, w1_ref[...], (((1,), (1,)), ((), ())),
                            preferred_element_type=jnp.float32,
                            precision=lax.Precision.DEFAULT)
    h = jnp.square(jnp.maximum(inter, 0.0))
    y_ref[...] = lax.dot_general(h, w2_ref[...], (((1,), (1,)), ((), ())),
                                 preferred_element_type=jnp.float32,
                                 precision=lax.Precision.DEFAULT)


def _shared_ffn(x, shared_w1, shared_w2):
    return pl.pallas_call(
        _shared_body,
        grid=(T // TILE,),
        in_specs=[
            pl.BlockSpec((TILE, D), lambda i: (i, 0)),
            pl.BlockSpec((M, D), lambda i: (0, 0)),
            pl.BlockSpec((D, M), lambda i: (0, 0)),
        ],
        out_specs=pl.BlockSpec((TILE, D), lambda i: (i, 0)),
        out_shape=jax.ShapeDtypeStruct((T, D), jnp.float32),
        compiler_params=pltpu.CompilerParams(
            dimension_semantics=("arbitrary",)),
    )(x, shared_w1, shared_w2)


def kernel(hidden_tensor, router_w, w1_stack, w2_stack, shared_w1, shared_w2):
    b, t, c = hidden_tensor.shape
    x = hidden_tensor.reshape(-1, c)

    pos1, pos2, g1, g2, tmap = _dispatch(x, router_w)
    x_sorted = _sc_scatter(x, pos1.reshape(-1), pos2.reshape(-1))
    shared = _shared_ffn(x, shared_w1, shared_w2)
    y_sorted = _expert_ffn(tmap, x_sorted, w1_stack, w2_stack)
    out = _sc_combine(y_sorted, pos1.reshape(-1), pos2.reshape(-1), g1, g2,
                      shared)
    return out.reshape(b, t, c)
